# Initial kernel scaffold; baseline (speedup 1.0000x reference)
import functools

import jax
import jax.numpy as jnp
from jax import lax
from jax.experimental import pallas as pl
from jax.experimental.pallas import tpu as pltpu
from jax.experimental.pallas import tpu_sc as plsc

N = 25000
D = 64
E = 400000
P = 100000
NC = 2
NS = 16

NPAD = 25088
EPAD = 401408
E_PER_TILE = EPAD // NS
PPAD = 100352
P_PER_TILE = PPAD // NS

ROWS_PER_TILE = NPAD // NS
RB = 200
NBLK = N // RB


def _dot(a, b):
    return jnp.dot(a, b, preferred_element_type=jnp.float32)


def _l2n(z):
    n = jnp.sqrt(jnp.sum(z * z, axis=1, keepdims=True))
    n = jnp.where(n == 0, jnp.ones_like(n), n)
    return z / n


def _tc_embed_body(huser_ref, hprod_ref, wue_ref, bue_ref, wie_ref, bie_ref,
                   wp_u_ref, wp_i_ref, hu_ref, hi_ref, m1_ref):
    hu = _dot(huser_ref[...], wue_ref[...]) + bue_ref[...]
    hi = _dot(hprod_ref[...], wie_ref[...]) + bie_ref[...]
    hu_ref[...] = hu
    hi_ref[...] = hi
    m1_ref[0] = jax.nn.relu(_dot(hu, wp_u_ref[...]))
    m1_ref[1] = jax.nn.relu(_dot(hi, wp_i_ref[...]))


def _tc_embed(h_user, h_product, W_user_emb, b_user_emb, W_item_emb,
              b_item_emb, W_pre1_up, W_pre1_pu):
    full = lambda shape: pl.BlockSpec(shape, lambda i: tuple(0 for _ in shape))
    return pl.pallas_call(
        _tc_embed_body,
        grid=(NBLK,),
        in_specs=[
            pl.BlockSpec((RB, 128), lambda i: (i, 0)),
            pl.BlockSpec((RB, 128), lambda i: (i, 0)),
            full((128, D)), full((1, D)), full((128, D)), full((1, D)),
            full((D, D)), full((D, D)),
        ],
        out_specs=[
            pl.BlockSpec((RB, D), lambda i: (i, 0)),
            pl.BlockSpec((RB, D), lambda i: (i, 0)),
            pl.BlockSpec((2, RB, D), lambda i: (0, i, 0)),
        ],
        out_shape=[
            jax.ShapeDtypeStruct((N, D), jnp.float32),
            jax.ShapeDtypeStruct((N, D), jnp.float32),
            jax.ShapeDtypeStruct((2, N, D), jnp.float32),
        ],
    )(h_user, h_product, W_user_emb, b_user_emb.reshape(1, D), W_item_emb,
      b_item_emb.reshape(1, D), W_pre1_up, W_pre1_pu)


def _make_tc_post(with_pre):
    def body(s_ref, cnt_ref, hu_ref, hi_ref, wsu_ref, wnu_ref, wsp_ref,
             wnp_ref, *rest):
        if with_pre:
            wp2u_ref, wp2p_ref, hu_out, hi_out, m2_ref = rest
        else:
            hu_out, hi_out = rest
        c_i = cnt_ref[0][:, 0:1]
        c_u = cnt_ref[1][:, 0:1]
        neigh_i = jnp.where(c_i > 0, s_ref[0] / jnp.where(c_i > 0, c_i, 1.0), 0.0)
        neigh_u = jnp.where(c_u > 0, s_ref[1] / jnp.where(c_u > 0, c_u, 1.0), 0.0)
        hi_n = _l2n(jax.nn.relu(_dot(hi_ref[...], wsu_ref[...])
                                + _dot(neigh_i, wnu_ref[...])))
        hu_n = _l2n(jax.nn.relu(_dot(hu_ref[...], wsp_ref[...])
                                + _dot(neigh_u, wnp_ref[...])))
        hu_out[...] = hu_n
        hi_out[...] = hi_n
        if with_pre:
            m2_ref[0] = jax.nn.relu(_dot(hu_n, wp2u_ref[...]))
            m2_ref[1] = jax.nn.relu(_dot(hi_n, wp2p_ref[...]))

    full = lambda shape: pl.BlockSpec(shape, lambda i: tuple(0 for _ in shape))
    in_specs = [
        pl.BlockSpec((2, RB, D), lambda i: (0, i, 0)),
        pl.BlockSpec((2, RB, 16), lambda i: (0, i, 0)),
        pl.BlockSpec((RB, D), lambda i: (i, 0)),
        pl.BlockSpec((RB, D), lambda i: (i, 0)),
        full((D, D)), full((D, D)), full((D, D)), full((D, D)),
    ]
    out_specs = [
        pl.BlockSpec((RB, D), lambda i: (i, 0)),
        pl.BlockSpec((RB, D), lambda i: (i, 0)),
    ]
    out_shape = [
        jax.ShapeDtypeStruct((N, D), jnp.float32),
        jax.ShapeDtypeStruct((N, D), jnp.float32),
    ]
    if with_pre:
        in_specs += [full((D, D)), full((D, D))]
        out_specs.append(pl.BlockSpec((2, RB, D), lambda i: (0, i, 0)))
        out_shape.append(jax.ShapeDtypeStruct((2, N, D), jnp.float32))

    def call(*args):
        return pl.pallas_call(body, grid=(NBLK,), in_specs=in_specs,
                              out_specs=out_specs, out_shape=out_shape)(*args)
    return call


def _zero_rows(ref, nrows, width):
    z = jnp.zeros((16,), jnp.float32)

    def row(r, _):
        for k in range(width // 16):
            ref[r, pl.ds(k * 16, 16)] = z
        return 0

    lax.fori_loop(0, nrows, row, 0)


def _make_segsum(with_counts):
    mesh = plsc.VectorSubcoreMesh(core_axis_name="c", subcore_axis_name="s")
    out_type = [jax.ShapeDtypeStruct((NC, NPAD, D), jnp.float32)]
    scratch = [
        pltpu.VMEM((4, 128), jnp.int32),
        pltpu.VMEM((4, 128), jnp.int32),
        pltpu.VMEM((512, D), jnp.float32),
        pltpu.VMEM((112, D), jnp.float32),
        pltpu.VMEM_SHARED((NPAD, D), jnp.float32),
        pltpu.SemaphoreType.DMA,
    ]
    if with_counts:
        out_type.append(jax.ShapeDtypeStruct((NC, NPAD, 16), jnp.float32))
        scratch += [
            pltpu.VMEM((128, 16), jnp.float32),
            pltpu.VMEM((392, 16), jnp.float32),
            pltpu.VMEM_SHARED((NPAD, 16), jnp.float32),
        ]

    def body(m_all, src_all, dst_all, *rest):
        if with_counts:
            (s_out, cnt_out, sidx, didx, msg, zbuf, acc, sem,
             ones_v, cbuf, cacc) = rest
        else:
            s_out, sidx, didx, msg, zbuf, acc, sem = rest
        cid = lax.axis_index("c")
        sid = lax.axis_index("s")

        _zero_rows(zbuf, 112, D)
        if with_counts:
            _zero_rows(cbuf, 392, 16)
            one = jnp.ones((16,), jnp.float32)

            def orow(r, _):
                ones_v[r, pl.ds(0, 16)] = one
                return 0
            lax.fori_loop(0, 128, orow, 0)

        row0 = sid * ROWS_PER_TILE

        def zloop(j, _):
            pltpu.sync_copy(zbuf, acc.at[pl.ds(row0 + j * 112, 112)])
            return 0
        lax.fori_loop(0, ROWS_PER_TILE // 112, zloop, 0)
        if with_counts:
            def czloop(j, _):
                pltpu.sync_copy(cbuf, cacc.at[pl.ds(row0 + j * 392, 392)])
                return 0
            lax.fori_loop(0, ROWS_PER_TILE // 392, czloop, 0)

        plsc.subcore_barrier()

        idx_row0 = sid * (E_PER_TILE // 128)

        def chunk(j, _):
            rb = idx_row0 + j * 4
            pltpu.sync_copy(src_all.at[cid, pl.ds(rb, 4)], sidx)
            pltpu.sync_copy(dst_all.at[cid, pl.ds(rb, 4)], didx)
            for q in range(4):
                pltpu.async_copy(m_all.at[cid].at[sidx.at[q]],
                                 msg.at[pl.ds(q * 128, 128)], sem)
            for q in range(4):
                pltpu.make_async_copy(m_all.at[cid].at[sidx.at[q]],
                                      msg.at[pl.ds(q * 128, 128)], sem).wait()
            for q in range(4):
                pltpu.sync_copy(msg.at[pl.ds(q * 128, 128)],
                                acc.at[didx.at[q]], add=True)
                if with_counts:
                    pltpu.sync_copy(ones_v, cacc.at[didx.at[q]], add=True)
            return 0
        lax.fori_loop(0, E_PER_TILE // 512, chunk, 0)

        plsc.subcore_barrier()

        def out_loop(j, _):
            rows = pl.ds(row0 + j * 112, 112)
            pltpu.sync_copy(acc.at[rows], zbuf)
            pltpu.sync_copy(zbuf, s_out.at[cid].at[rows])
            return 0
        lax.fori_loop(0, ROWS_PER_TILE // 112, out_loop, 0)
        if with_counts:
            def cout_loop(j, _):
                rows = pl.ds(row0 + j * 392, 392)
                pltpu.sync_copy(cacc.at[rows], cbuf)
                pltpu.sync_copy(cbuf, cnt_out.at[cid].at[rows])
                return 0
            lax.fori_loop(0, ROWS_PER_TILE // 392, cout_loop, 0)

    def call(m_all, src_all, dst_all):
        return pl.kernel(body, out_type=out_type, mesh=mesh,
                         scratch_types=scratch)(m_all, src_all, dst_all)
    return call


def _scores_body(hu2, hi2, pu_all, pi_all, sc_out,
                 uidx, vidx, urows, vrows, tbuf, sbuf, sem):
    cid = lax.axis_index("c")
    sid = lax.axis_index("s")
    idx_row0 = sid * (P_PER_TILE // 128)
    lanes = lax.iota(jnp.int32, 16)

    def chunk(j, _):
        rb = idx_row0 + j
        pltpu.sync_copy(pu_all.at[cid, pl.ds(rb, 1)], uidx)
        pltpu.sync_copy(pi_all.at[cid, pl.ds(rb, 1)], vidx)
        pltpu.async_copy(hu2.at[uidx.at[0]], urows, sem)
        pltpu.async_copy(hi2.at[vidx.at[0]], vrows, sem)
        pltpu.make_async_copy(hu2.at[uidx.at[0]], urows, sem).wait()
        pltpu.make_async_copy(hi2.at[vidx.at[0]], vrows, sem).wait()
        for g in range(8):
            for p in range(16):
                r = g * 16 + p
                acc = (urows[r, pl.ds(0, 16)] * vrows[r, pl.ds(0, 16)]
                       + urows[r, pl.ds(16, 16)] * vrows[r, pl.ds(16, 16)]
                       + urows[r, pl.ds(32, 16)] * vrows[r, pl.ds(32, 16)]
                       + urows[r, pl.ds(48, 16)] * vrows[r, pl.ds(48, 16)])
                plsc.store_scatter(tbuf, [lanes, jnp.full((16,), p, jnp.int32)],
                                   acc)
            tot = tbuf[0, pl.ds(0, 16)]
            for rr in range(1, 16):
                tot = tot + tbuf[rr, pl.ds(0, 16)]
            sbuf[pl.ds(g * 16, 16)] = tot
        pltpu.sync_copy(sbuf, sc_out.at[cid, pl.ds(rb * 128, 128)])
        return 0
    lax.fori_loop(0, P_PER_TILE // 128, chunk, 0)


def _scores_call(hu2, hi2, pu_all, pi_all):
    mesh = plsc.VectorSubcoreMesh(core_axis_name="c", subcore_axis_name="s")
    return pl.kernel(
        _scores_body,
        out_type=jax.ShapeDtypeStruct((NC, PPAD), jnp.float32),
        mesh=mesh,
        scratch_types=[
            pltpu.VMEM((1, 128), jnp.int32),
            pltpu.VMEM((1, 128), jnp.int32),
            pltpu.VMEM((128, D), jnp.float32),
            pltpu.VMEM((128, D), jnp.float32),
            pltpu.VMEM((16, 16), jnp.float32),
            pltpu.VMEM((128,), jnp.float32),
            pltpu.SemaphoreType.DMA,
        ],
    )(hu2, hi2, pu_all, pi_all)


_segsum_with_counts = _make_segsum(True)
_segsum_no_counts = _make_segsum(False)
_tc_post_pre = _make_tc_post(True)
_tc_post_final = _make_tc_post(False)


def _pad_idx(a, value, total):
    a = a.astype(jnp.int32)
    return jnp.concatenate(
        [a, jnp.full((total - a.shape[0],), value, jnp.int32)])


def kernel(h_user, h_product, edge_u, edge_i, pos_u, pos_i, neg_u, neg_i,
           W_user_emb, b_user_emb, W_item_emb, b_item_emb,
           W_pre1_up, W_neigh1_up, W_self1_up,
           W_pre1_pu, W_neigh1_pu, W_self1_pu,
           W_pre2_up, W_neigh2_up, W_self2_up,
           W_pre2_pu, W_neigh2_pu, W_self2_pu):
    hu, hi, m1 = _tc_embed(h_user, h_product, W_user_emb, b_user_emb,
                           W_item_emb, b_item_emb, W_pre1_up, W_pre1_pu)

    src_all = jnp.stack([_pad_idx(edge_u, 0, EPAD),
                         _pad_idx(edge_i, 0, EPAD)]).reshape(NC, EPAD // 128, 128)
    dst_all = jnp.stack([_pad_idx(edge_i, N, EPAD),
                         _pad_idx(edge_u, N, EPAD)]).reshape(NC, EPAD // 128, 128)

    s1, cnt = _segsum_with_counts(m1, src_all, dst_all)
    hu1, hi1, m2 = _tc_post_pre(s1, cnt, hu, hi,
                                W_self1_up, W_neigh1_up,
                                W_self1_pu, W_neigh1_pu,
                                W_pre2_up, W_pre2_pu)
    s2 = _segsum_no_counts(m2, src_all, dst_all)
    hu2, hi2 = _tc_post_final(s2, cnt, hu1, hi1,
                              W_self2_up, W_neigh2_up,
                              W_self2_pu, W_neigh2_pu)

    pu_all = jnp.stack([_pad_idx(pos_u, 0, PPAD),
                        _pad_idx(neg_u, 0, PPAD)]).reshape(NC, PPAD // 128, 128)
    pi_all = jnp.stack([_pad_idx(pos_i, 0, PPAD),
                        _pad_idx(neg_i, 0, PPAD)]).reshape(NC, PPAD // 128, 128)
    sc = _scores_call(hu2, hi2, pu_all, pi_all)
    return hu2, hi2, sc[0, :P], sc[1, :P]


# speedup vs baseline: 5.4766x; 5.4766x over previous
import jax
import jax.numpy as jnp
from jax import lax
from jax.experimental import pallas as pl
from jax.experimental.pallas import tpu as pltpu
from jax.experimental.pallas import tpu_sc as plsc

N = 25000
D = 64
DH = 32
E = 400000
P = 100000
NC = 2
NS = 16

NPAD = 25088
EPAD = 401408
E_PER_TILE = EPAD // NS
PPAD = 100352
P_PER_TILE = PPAD // NS

ROWS_PER_TILE = NPAD // NS
RB = 200
NBLK = N // RB


def _dot(a, b):
    return jnp.dot(a, b, preferred_element_type=jnp.float32)


def _l2n(z):
    n = jnp.sqrt(jnp.sum(z * z, axis=1, keepdims=True))
    n = jnp.where(n == 0, jnp.ones_like(n), n)
    return z / n


def _tc_embed_body(huser_ref, hprod_ref, wue_ref, bue_ref, wie_ref, bie_ref,
                   wp_u_ref, wp_i_ref, hu_ref, hi_ref, mlo_ref, mhi_ref):
    hu = _dot(huser_ref[...], wue_ref[...]) + bue_ref[...]
    hi = _dot(hprod_ref[...], wie_ref[...]) + bie_ref[...]
    hu_ref[...] = hu
    hi_ref[...] = hi
    mu = jax.nn.relu(_dot(hu, wp_u_ref[...]))
    mi = jax.nn.relu(_dot(hi, wp_i_ref[...]))
    mlo_ref[0] = mu[:, :DH]
    mhi_ref[0] = mu[:, DH:]
    mlo_ref[1] = mi[:, :DH]
    mhi_ref[1] = mi[:, DH:]


def _tc_embed(h_user, h_product, W_user_emb, b_user_emb, W_item_emb,
              b_item_emb, W_pre1_up, W_pre1_pu):
    full = lambda shape: pl.BlockSpec(shape, lambda i: tuple(0 for _ in shape))
    return pl.pallas_call(
        _tc_embed_body,
        grid=(NBLK,),
        in_specs=[
            pl.BlockSpec((RB, 128), lambda i: (i, 0)),
            pl.BlockSpec((RB, 128), lambda i: (i, 0)),
            full((128, D)), full((1, D)), full((128, D)), full((1, D)),
            full((D, D)), full((D, D)),
        ],
        out_specs=[
            pl.BlockSpec((RB, D), lambda i: (i, 0)),
            pl.BlockSpec((RB, D), lambda i: (i, 0)),
            pl.BlockSpec((2, RB, DH), lambda i: (0, i, 0)),
            pl.BlockSpec((2, RB, DH), lambda i: (0, i, 0)),
        ],
        out_shape=[
            jax.ShapeDtypeStruct((N, D), jnp.float32),
            jax.ShapeDtypeStruct((N, D), jnp.float32),
            jax.ShapeDtypeStruct((2, N, DH), jnp.float32),
            jax.ShapeDtypeStruct((2, N, DH), jnp.float32),
        ],
    )(h_user, h_product, W_user_emb, b_user_emb.reshape(1, D), W_item_emb,
      b_item_emb.reshape(1, D), W_pre1_up, W_pre1_pu)


def _make_tc_post(with_pre):
    def body(slo_ref, shi_ref, cnt_ref, hu_ref, hi_ref, wsu_ref, wnu_ref,
             wsp_ref, wnp_ref, *rest):
        if with_pre:
            wp2u_ref, wp2p_ref, hu_out, hi_out, mlo_ref, mhi_ref = rest
        else:
            hu_out, hi_out = rest
        s_i = jnp.concatenate([slo_ref[0, 0], shi_ref[0, 0]], axis=1)
        s_u = jnp.concatenate([slo_ref[1, 0], shi_ref[1, 0]], axis=1)
        c_i = cnt_ref[0][:, 0:1]
        c_u = cnt_ref[1][:, 0:1]
        neigh_i = jnp.where(c_i > 0, s_i / jnp.where(c_i > 0, c_i, 1.0), 0.0)
        neigh_u = jnp.where(c_u > 0, s_u / jnp.where(c_u > 0, c_u, 1.0), 0.0)
        hi_n = _l2n(jax.nn.relu(_dot(hi_ref[...], wsu_ref[...])
                                + _dot(neigh_i, wnu_ref[...])))
        hu_n = _l2n(jax.nn.relu(_dot(hu_ref[...], wsp_ref[...])
                                + _dot(neigh_u, wnp_ref[...])))
        hu_out[...] = hu_n
        hi_out[...] = hi_n
        if with_pre:
            m2u = jax.nn.relu(_dot(hu_n, wp2u_ref[...]))
            m2i = jax.nn.relu(_dot(hi_n, wp2p_ref[...]))
            mlo_ref[0] = m2u[:, :DH]
            mhi_ref[0] = m2u[:, DH:]
            mlo_ref[1] = m2i[:, :DH]
            mhi_ref[1] = m2i[:, DH:]

    full = lambda shape: pl.BlockSpec(shape, lambda i: tuple(0 for _ in shape))
    in_specs = [
        pl.BlockSpec((2, 1, RB, DH), lambda i: (0, 0, i, 0)),
        pl.BlockSpec((2, 1, RB, DH), lambda i: (0, 1, i, 0)),
        pl.BlockSpec((2, RB, 16), lambda i: (0, i, 0)),
        pl.BlockSpec((RB, D), lambda i: (i, 0)),
        pl.BlockSpec((RB, D), lambda i: (i, 0)),
        full((D, D)), full((D, D)), full((D, D)), full((D, D)),
    ]
    out_specs = [
        pl.BlockSpec((RB, D), lambda i: (i, 0)),
        pl.BlockSpec((RB, D), lambda i: (i, 0)),
    ]
    out_shape = [
        jax.ShapeDtypeStruct((N, D), jnp.float32),
        jax.ShapeDtypeStruct((N, D), jnp.float32),
    ]
    if with_pre:
        in_specs += [full((D, D)), full((D, D))]
        out_specs += [pl.BlockSpec((2, RB, DH), lambda i: (0, i, 0)),
                      pl.BlockSpec((2, RB, DH), lambda i: (0, i, 0))]
        out_shape += [jax.ShapeDtypeStruct((2, N, DH), jnp.float32),
                      jax.ShapeDtypeStruct((2, N, DH), jnp.float32)]

    def call(s4, cnt, *args):
        return pl.pallas_call(body, grid=(NBLK,), in_specs=in_specs,
                              out_specs=out_specs, out_shape=out_shape)(
                                  s4, s4, cnt, *args)
    return call


def _zero_rows(ref, nrows, width):
    z = jnp.zeros((16,), jnp.float32)

    def row(r, _):
        for k in range(width // 16):
            ref[r, pl.ds(k * 16, 16)] = z
        return 0

    lax.fori_loop(0, nrows, row, 0)


def _sc_mesh():
    return plsc.VectorSubcoreMesh(core_axis_name="c", subcore_axis_name="s",
                                  num_cores=NC, num_subcores=NS)


def _sc_params():
    return pltpu.CompilerParams(use_tc_tiling_on_sc=False,
                                needs_layout_passes=False)


def _segsum_body(m_lo, m_hi, src_all, dst_all, s_out,
                 sidx, didx, msg, zbuf, acc, sem):
    cid = lax.axis_index("c")
    sid = lax.axis_index("s")

    _zero_rows(zbuf, 112, DH)
    row0 = sid * ROWS_PER_TILE
    idx_row0 = sid * (E_PER_TILE // 128)

    for h, m_tab in ((0, m_lo), (1, m_hi)):
        def zloop(j, _):
            pltpu.sync_copy(zbuf, acc.at[pl.ds(row0 + j * 112, 112)])
            return 0
        lax.fori_loop(0, ROWS_PER_TILE // 112, zloop, 0)

        plsc.subcore_barrier()

        def chunk(j, _):
            rb = idx_row0 + j * 4
            pltpu.sync_copy(src_all.at[cid, pl.ds(rb, 4)], sidx)
            pltpu.sync_copy(dst_all.at[cid, pl.ds(rb, 4)], didx)
            for q in range(4):
                pltpu.async_copy(m_tab.at[cid].at[sidx.at[q]],
                                 msg.at[pl.ds(q * 128, 128)], sem)
            for q in range(4):
                pltpu.make_async_copy(m_tab.at[cid].at[sidx.at[q]],
                                      msg.at[pl.ds(q * 128, 128)], sem).wait()
            for q in range(4):
                pltpu.sync_copy(msg.at[pl.ds(q * 128, 128)],
                                acc.at[didx.at[q]], add=True)
            return 0
        lax.fori_loop(0, E_PER_TILE // 512, chunk, 0)

        plsc.subcore_barrier()

        def out_loop(j, _):
            rows = pl.ds(row0 + j * 112, 112)
            pltpu.sync_copy(acc.at[rows], zbuf)
            pltpu.sync_copy(zbuf, s_out.at[cid, h].at[rows])
            _zero_rows(zbuf, 112, DH)
            return 0
        lax.fori_loop(0, ROWS_PER_TILE // 112, out_loop, 0)


def _segsum_call(m_lo, m_hi, src_all, dst_all):
    return pl.kernel(
        _segsum_body,
        out_type=jax.ShapeDtypeStruct((NC, 2, NPAD, DH), jnp.float32),
        mesh=_sc_mesh(),
        compiler_params=_sc_params(),
        scratch_types=[
            pltpu.VMEM((4, 128), jnp.int32),
            pltpu.VMEM((4, 128), jnp.int32),
            pltpu.VMEM((512, DH), jnp.float32),
            pltpu.VMEM((112, DH), jnp.float32),
            pltpu.VMEM_SHARED((NPAD, DH), jnp.float32),
            pltpu.SemaphoreType.DMA,
        ],
    )(m_lo, m_hi, src_all, dst_all)


def _counts_body(dst_all, cnt_out, didx, ones_v, cbuf, cacc, sem):
    del sem
    cid = lax.axis_index("c")
    sid = lax.axis_index("s")

    _zero_rows(cbuf, 392, 16)
    one = jnp.ones((16,), jnp.float32)

    def orow(r, _):
        ones_v[r, pl.ds(0, 16)] = one
        return 0
    lax.fori_loop(0, 128, orow, 0)

    row0 = sid * ROWS_PER_TILE

    def czloop(j, _):
        pltpu.sync_copy(cbuf, cacc.at[pl.ds(row0 + j * 392, 392)])
        return 0
    lax.fori_loop(0, ROWS_PER_TILE // 392, czloop, 0)

    plsc.subcore_barrier()

    idx_row0 = sid * (E_PER_TILE // 128)

    def chunk(j, _):
        rb = idx_row0 + j * 4
        pltpu.sync_copy(dst_all.at[cid, pl.ds(rb, 4)], didx)
        for q in range(4):
            pltpu.sync_copy(ones_v, cacc.at[didx.at[q]], add=True)
        return 0
    lax.fori_loop(0, E_PER_TILE // 512, chunk, 0)

    plsc.subcore_barrier()

    def cout_loop(j, _):
        rows = pl.ds(row0 + j * 392, 392)
        pltpu.sync_copy(cacc.at[rows], cbuf)
        pltpu.sync_copy(cbuf, cnt_out.at[cid].at[rows])
        return 0
    lax.fori_loop(0, ROWS_PER_TILE // 392, cout_loop, 0)


def _counts_call(dst_all):
    return pl.kernel(
        _counts_body,
        out_type=jax.ShapeDtypeStruct((NC, NPAD, 16), jnp.float32),
        mesh=_sc_mesh(),
        compiler_params=_sc_params(),
        scratch_types=[
            pltpu.VMEM((4, 128), jnp.int32),
            pltpu.VMEM((128, 16), jnp.float32),
            pltpu.VMEM((392, 16), jnp.float32),
            pltpu.VMEM_SHARED((NPAD, 16), jnp.float32),
            pltpu.SemaphoreType.DMA,
        ],
    )(dst_all)


def _scores_body(hu2, hi2, pu_all, pi_all, sc_out,
                 uidx, vidx, urows, vrows, tbuf, sbuf, sem):
    cid = lax.axis_index("c")
    sid = lax.axis_index("s")
    idx_row0 = sid * (P_PER_TILE // 128)
    lanes = lax.iota(jnp.int32, 16)

    def chunk(j, _):
        rb = idx_row0 + j
        pltpu.sync_copy(pu_all.at[cid, pl.ds(rb, 1)], uidx)
        pltpu.sync_copy(pi_all.at[cid, pl.ds(rb, 1)], vidx)
        pltpu.async_copy(hu2.at[uidx.at[0]], urows, sem)
        pltpu.async_copy(hi2.at[vidx.at[0]], vrows, sem)
        pltpu.make_async_copy(hu2.at[uidx.at[0]], urows, sem).wait()
        pltpu.make_async_copy(hi2.at[vidx.at[0]], vrows, sem).wait()
        for g in range(8):
            for p in range(16):
                r = g * 16 + p
                acc = (urows[r, pl.ds(0, 16)] * vrows[r, pl.ds(0, 16)]
                       + urows[r, pl.ds(16, 16)] * vrows[r, pl.ds(16, 16)]
                       + urows[r, pl.ds(32, 16)] * vrows[r, pl.ds(32, 16)]
                       + urows[r, pl.ds(48, 16)] * vrows[r, pl.ds(48, 16)])
                plsc.store_scatter(tbuf, [lanes, jnp.full((16,), p, jnp.int32)],
                                   acc)
            tot = tbuf[0, pl.ds(0, 16)]
            for rr in range(1, 16):
                tot = tot + tbuf[rr, pl.ds(0, 16)]
            sbuf[pl.ds(g * 16, 16)] = tot
        pltpu.sync_copy(sbuf, sc_out.at[cid, pl.ds(rb * 128, 128)])
        return 0
    lax.fori_loop(0, P_PER_TILE // 128, chunk, 0)


def _scores_call(hu2, hi2, pu_all, pi_all):
    return pl.kernel(
        _scores_body,
        out_type=jax.ShapeDtypeStruct((NC, PPAD), jnp.float32),
        mesh=_sc_mesh(),
        compiler_params=_sc_params(),
        scratch_types=[
            pltpu.VMEM((1, 128), jnp.int32),
            pltpu.VMEM((1, 128), jnp.int32),
            pltpu.VMEM((128, D), jnp.float32),
            pltpu.VMEM((128, D), jnp.float32),
            pltpu.VMEM((16, 16), jnp.float32),
            pltpu.VMEM((128,), jnp.float32),
            pltpu.SemaphoreType.DMA,
        ],
    )(hu2, hi2, pu_all, pi_all)


def _pad_idx(a, value, total):
    a = a.astype(jnp.int32)
    return jnp.concatenate(
        [a, jnp.full((total - a.shape[0],), value, jnp.int32)])


_tc_post_pre = _make_tc_post(True)
_tc_post_final = _make_tc_post(False)


def kernel(h_user, h_product, edge_u, edge_i, pos_u, pos_i, neg_u, neg_i,
           W_user_emb, b_user_emb, W_item_emb, b_item_emb,
           W_pre1_up, W_neigh1_up, W_self1_up,
           W_pre1_pu, W_neigh1_pu, W_self1_pu,
           W_pre2_up, W_neigh2_up, W_self2_up,
           W_pre2_pu, W_neigh2_pu, W_self2_pu):
    hu, hi, m1_lo, m1_hi = _tc_embed(h_user, h_product, W_user_emb,
                                     b_user_emb, W_item_emb, b_item_emb,
                                     W_pre1_up, W_pre1_pu)

    src_all = jnp.stack([_pad_idx(edge_u, 0, EPAD),
                         _pad_idx(edge_i, 0, EPAD)]).reshape(NC, EPAD // 128, 128)
    dst_all = jnp.stack([_pad_idx(edge_i, N, EPAD),
                         _pad_idx(edge_u, N, EPAD)]).reshape(NC, EPAD // 128, 128)

    cnt = _counts_call(dst_all)
    s1 = _segsum_call(m1_lo, m1_hi, src_all, dst_all)
    hu1, hi1, m2_lo, m2_hi = _tc_post_pre(s1, cnt, hu, hi,
                                          W_self1_up, W_neigh1_up,
                                          W_self1_pu, W_neigh1_pu,
                                          W_pre2_up, W_pre2_pu)
    s2 = _segsum_call(m2_lo, m2_hi, src_all, dst_all)
    hu2, hi2 = _tc_post_final(s2, cnt, hu1, hi1,
                              W_self2_up, W_neigh2_up,
                              W_self2_pu, W_neigh2_pu)

    pu_all = jnp.stack([_pad_idx(pos_u, 0, PPAD),
                        _pad_idx(neg_u, 0, PPAD)]).reshape(NC, PPAD // 128, 128)
    pi_all = jnp.stack([_pad_idx(pos_i, 0, PPAD),
                        _pad_idx(neg_i, 0, PPAD)]).reshape(NC, PPAD // 128, 128)
    sc = _scores_call(hu2, hi2, pu_all, pi_all)
    return hu2, hi2, sc[0, :P], sc[1, :P]
